# Initial kernel scaffold; baseline (speedup 1.0000x reference)
import jax
import jax.numpy as jnp
from jax.experimental import pallas as pl


def kernel(x, edges, p1_W, p1_b, l1_Wl, l1_bl, l1_Wr, p2_W, p2_b, l2_Wl, l2_bl, l2_Wr, p3_W, p3_b, l3_Wl, l3_bl, l3_Wr, fc1_W, fc1_b, fc2_W, fc2_b, fc3_W, fc3_b):
    raise NotImplementedError("write your pallas kernel here")


# speedup vs baseline: 5.1330x; 5.1330x over previous
import functools

import jax
import jax.numpy as jnp
from jax import lax
from jax.experimental import pallas as pl
from jax.experimental.pallas import tpu as pltpu
from jax.experimental.pallas import tpu_sc as plsc

_NUM_CORES = 2
_NUM_SUBCORES = 16
_EDGE_BATCH = 128
_DH = 256
_HALF = 128


def _rup(v, m):
  return (v + m - 1) // m * m


def _mm_body(a_ref, b_ref, bias_ref, o_ref, *, act):
  r = jnp.dot(a_ref[...], b_ref[...], preferred_element_type=jnp.float32)
  r = r + bias_ref[...]
  if act == "relu":
    r = jnp.maximum(r, 0.0)
  elif act == "elu":
    r = jnp.where(r > 0.0, r, jnp.exp(jnp.minimum(r, 0.0)) - 1.0)
  o_ref[...] = r


def _mm(a, b, bias, act, bm, bn):
  m, k = a.shape
  _, n = b.shape
  bm = min(bm, m)
  bn = min(bn, n)
  assert m % bm == 0 and n % bn == 0, (m, n, bm, bn)
  grid = (n // bn, m // bm)
  return pl.pallas_call(
      functools.partial(_mm_body, act=act),
      grid=grid,
      in_specs=[
          pl.BlockSpec((bm, k), lambda j, i: (i, 0)),
          pl.BlockSpec((k, bn), lambda j, i: (0, j)),
          pl.BlockSpec((1, bn), lambda j, i: (0, j)),
      ],
      out_specs=pl.BlockSpec((bm, bn), lambda j, i: (i, j)),
      out_shape=jax.ShapeDtypeStruct((m, n), jnp.float32),
      compiler_params=pltpu.CompilerParams(
          dimension_semantics=("parallel", "parallel")
      ),
  )(a, b, bias.reshape(1, n))


def _ep_body(alo_ref, ahi_ref, cnt_ref, r_ref, bl_ref, o_ref):
  t = jnp.concatenate([alo_ref[...], ahi_ref[...]], axis=1)
  inv = 1.0 / jnp.maximum(cnt_ref[:, 0:1], 1.0)
  t = t * inv + bl_ref[...] + r_ref[...]
  nrm = jnp.sqrt(jnp.sum(t * t, axis=1, keepdims=True))
  t = t / jnp.maximum(nrm, 1e-12)
  o_ref[...] = jnp.where(t > 0.0, t, jnp.exp(jnp.minimum(t, 0.0)) - 1.0)


def _sage_epilogue(agg_lo, agg_hi, cnt, r, bl, bm):
  m = r.shape[0]
  return pl.pallas_call(
      _ep_body,
      grid=(m // bm,),
      in_specs=[
          pl.BlockSpec((bm, _HALF), lambda i: (i, 0)),
          pl.BlockSpec((bm, _HALF), lambda i: (i, 0)),
          pl.BlockSpec((bm, 16), lambda i: (i, 0)),
          pl.BlockSpec((bm, _DH), lambda i: (i, 0)),
          pl.BlockSpec((1, _DH), lambda i: (0, 0)),
      ],
      out_specs=pl.BlockSpec((bm, _DH), lambda i: (i, 0)),
      out_shape=jax.ShapeDtypeStruct((m, _DH), jnp.float32),
      compiler_params=pltpu.CompilerParams(
          dimension_semantics=("parallel",)
      ),
  )(agg_lo, agg_hi, cnt, r, bl.reshape(1, _DH))


def _sc_mesh():
  return plsc.VectorSubcoreMesh(
      core_axis_name="c",
      subcore_axis_name="s",
      num_cores=_NUM_CORES,
      num_subcores=_NUM_SUBCORES,
  )


def _make_sc_agg(np_, nb):
  rows_per = np_ // _NUM_SUBCORES

  def body(y_lo, y_hi, src_t, dst_t, zrows, agg_lo, agg_hi,
           src_v, dst_v, rows_v, acc_sh, sem):
    c = lax.axis_index("c")
    s = lax.axis_index("s")
    sl = pl.ds(s * rows_per, rows_per)

    pltpu.sync_copy(zrows, acc_sh.at[sl])
    pltpu.sync_copy(src_t.at[s], src_v)
    pltpu.sync_copy(dst_t.at[s], dst_v)
    plsc.subcore_barrier()

    def run(y_ref):
      def step(j, carry):
        pltpu.async_copy(y_ref.at[src_v.at[j]], rows_v, sem).wait()
        pltpu.sync_copy(rows_v, acc_sh.at[dst_v.at[j]], add=True)
        return carry
      lax.fori_loop(0, nb, step, 0)

    @pl.when(c == 0)
    def _():
      run(y_lo)

    @pl.when(c == 1)
    def _():
      run(y_hi)

    plsc.subcore_barrier()

    @pl.when(c == 0)
    def _():
      pltpu.sync_copy(acc_sh.at[sl], agg_lo.at[sl])

    @pl.when(c == 1)
    def _():
      pltpu.sync_copy(acc_sh.at[sl], agg_hi.at[sl])

  return pl.kernel(
      body,
      out_type=[jax.ShapeDtypeStruct((np_, _HALF), jnp.float32)] * 2,
      mesh=_sc_mesh(),
      scratch_types=[
          pltpu.VMEM((nb, _EDGE_BATCH), jnp.int32),
          pltpu.VMEM((nb, _EDGE_BATCH), jnp.int32),
          pltpu.VMEM((_EDGE_BATCH, _HALF), jnp.float32),
          pltpu.VMEM_SHARED((np_, _HALF), jnp.float32),
          pltpu.SemaphoreType.DMA,
      ],
  )


def _make_sc_counts(np_, nb):
  rows_per = np_ // _NUM_SUBCORES

  def body(dst_t, zrows, ones_r, cnt, dst_v, ones_v, cnt_sh):
    c = lax.axis_index("c")
    s = lax.axis_index("s")
    sl = pl.ds(s * rows_per, rows_per)

    @pl.when(c == 0)
    def _():
      pltpu.sync_copy(zrows, cnt_sh.at[sl])
      pltpu.sync_copy(dst_t.at[s], dst_v)
      pltpu.sync_copy(ones_r, ones_v)
    plsc.subcore_barrier()

    @pl.when(c == 0)
    def _():
      def cstep(j, carry):
        pltpu.sync_copy(ones_v, cnt_sh.at[dst_v.at[j]], add=True)
        return carry
      lax.fori_loop(0, nb, cstep, 0)
    plsc.subcore_barrier()

    @pl.when(c == 0)
    def _():
      pltpu.sync_copy(cnt_sh.at[sl], cnt.at[sl])

  return pl.kernel(
      body,
      out_type=[jax.ShapeDtypeStruct((np_, _HALF), jnp.float32)],
      mesh=_sc_mesh(),
      scratch_types=[
          pltpu.VMEM((nb, _EDGE_BATCH), jnp.int32),
          pltpu.VMEM((_EDGE_BATCH, _HALF), jnp.float32),
          pltpu.VMEM_SHARED((np_, _HALF), jnp.float32),
      ],
  )


def kernel(x, edges, p1_W, p1_b, l1_Wl, l1_bl, l1_Wr,
           p2_W, p2_b, l2_Wl, l2_bl, l2_Wr,
           p3_W, p3_b, l3_Wl, l3_bl, l3_Wr,
           fc1_W, fc1_b, fc2_W, fc2_b, fc3_W, fc3_b):
  f32 = jnp.float32
  n, d_in = x.shape
  e = edges.shape[1]
  np_ = _rup(n + 1, 2048)
  dp = _rup(d_in, 128)
  d_out = fc3_W.shape[1]

  xp_in = jnp.pad(x, ((0, np_ - n), (0, dp - d_in)))
  p1_Wp = jnp.pad(p1_W, ((0, dp - d_in), (0, dp - d_in)))
  p1_bp = jnp.pad(p1_b, (0, dp - d_in))
  l1_Wlp = jnp.pad(l1_Wl, ((0, dp - d_in), (0, 0)))
  l1_Wrp = jnp.pad(l1_Wr, ((0, dp - d_in), (0, 0)))

  chunk = _NUM_SUBCORES * _EDGE_BATCH
  nb = _rup(e, chunk) // chunk
  ep = nb * chunk
  src = jnp.concatenate([edges[0], jnp.zeros((ep - e,), jnp.int32)])
  dst = jnp.concatenate([edges[1], jnp.full((ep - e,), n, jnp.int32)])
  src_t = src.reshape(_NUM_SUBCORES, nb, _EDGE_BATCH)
  dst_t = dst.reshape(_NUM_SUBCORES, nb, _EDGE_BATCH)

  rows_per = np_ // _NUM_SUBCORES
  zrows = jnp.zeros((rows_per, _HALF), f32)
  ones_r = jnp.ones((_EDGE_BATCH, _HALF), f32)
  zb = jnp.zeros((_DH,), f32)

  sc_agg = _make_sc_agg(np_, nb)
  sc_counts = _make_sc_counts(np_, nb)

  cnt = sc_counts(dst_t, zrows, ones_r)
  if isinstance(cnt, (list, tuple)):
    cnt = cnt[0]
  cnt = cnt[:, :16]
  xp1 = _mm(xp_in, p1_Wp, p1_bp, "relu", bm=512, bn=896)
  y1 = _mm(xp1, l1_Wlp, zb, None, bm=1024, bn=_DH)
  r1 = _mm(xp_in, l1_Wrp, zb, None, bm=1024, bn=_DH)
  agg_lo, agg_hi = sc_agg(
      y1[:, :_HALF], y1[:, _HALF:], src_t, dst_t, zrows)
  h = _sage_epilogue(agg_lo, agg_hi, cnt, r1, l1_bl, bm=1024)

  for pw, pb, wl, bl, wr in (
      (p2_W, p2_b, l2_Wl, l2_bl, l2_Wr),
      (p3_W, p3_b, l3_Wl, l3_bl, l3_Wr),
  ):
    xpk = _mm(h, pw, pb, "relu", bm=1024, bn=_DH)
    yk = _mm(xpk, wl, zb, None, bm=1024, bn=_DH)
    rk = _mm(h, wr, zb, None, bm=1024, bn=_DH)
    a_lo, a_hi = sc_agg(yk[:, :_HALF], yk[:, _HALF:], src_t, dst_t, zrows)
    h = _sage_epilogue(a_lo, a_hi, cnt, rk, bl, bm=1024)

  h = _mm(h, fc1_W, fc1_b, "elu", bm=1024, bn=_DH)
  h = _mm(h, fc2_W, fc2_b, "elu", bm=1024, bn=_DH)
  do_p = _rup(d_out, 128)
  fc3_Wp = jnp.pad(fc3_W, ((0, 0), (0, do_p - d_out)))
  fc3_bp = jnp.pad(fc3_b, (0, do_p - d_out))
  out = _mm(h, fc3_Wp, fc3_bp, None, bm=1024, bn=do_p)
  return out[:n, :d_out]


# speedup vs baseline: 5.3884x; 1.0498x over previous
import functools

import jax
import jax.numpy as jnp
from jax import lax
from jax.experimental import pallas as pl
from jax.experimental.pallas import tpu as pltpu
from jax.experimental.pallas import tpu_sc as plsc

_NUM_CORES = 2
_NUM_SUBCORES = 16
_EDGE_BATCH = 128
_DH = 256
_HALF = 128


def _rup(v, m):
  return (v + m - 1) // m * m


def _l1_body(x_ref, w_ref, b_ref, wl_ref, wr_ref,
             ylo_ref, yhi_ref, r_ref, xp_ref):
  dp = x_ref.shape[1]
  kt = 896 if dp % 896 == 0 else dp
  for t in range(dp // kt):
    sl = slice(t * kt, (t + 1) * kt)
    xp_ref[:, sl] = jnp.maximum(
        jnp.dot(x_ref[...], w_ref[:, sl],
                preferred_element_type=jnp.float32) + b_ref[:, sl], 0.0)
  y = jnp.dot(xp_ref[...], wl_ref[...], preferred_element_type=jnp.float32)
  ylo_ref[...] = y[:, :_HALF]
  yhi_ref[...] = y[:, _HALF:]
  r_ref[...] = jnp.dot(x_ref[...], wr_ref[...],
                       preferred_element_type=jnp.float32)


def _l1_dense(x, w, b, wl, wr, bm):
  m, dp = x.shape
  return pl.pallas_call(
      _l1_body,
      grid=(m // bm,),
      in_specs=[
          pl.BlockSpec((bm, dp), lambda i: (i, 0)),
          pl.BlockSpec((dp, dp), lambda i: (0, 0)),
          pl.BlockSpec((1, dp), lambda i: (0, 0)),
          pl.BlockSpec((dp, _DH), lambda i: (0, 0)),
          pl.BlockSpec((dp, _DH), lambda i: (0, 0)),
      ],
      out_specs=[
          pl.BlockSpec((bm, _HALF), lambda i: (i, 0)),
          pl.BlockSpec((bm, _HALF), lambda i: (i, 0)),
          pl.BlockSpec((bm, _DH), lambda i: (i, 0)),
      ],
      out_shape=[
          jax.ShapeDtypeStruct((m, _HALF), jnp.float32),
          jax.ShapeDtypeStruct((m, _HALF), jnp.float32),
          jax.ShapeDtypeStruct((m, _DH), jnp.float32),
      ],
      scratch_shapes=[pltpu.VMEM((bm, dp), jnp.float32)],
      compiler_params=pltpu.CompilerParams(
          dimension_semantics=("parallel",)
      ),
  )(x, w, b.reshape(1, dp), wl, wr)


def _epilogue_h(alo_ref, ahi_ref, cnt_ref, r_ref, bl_ref):
  t = jnp.concatenate([alo_ref[...], ahi_ref[...]], axis=1)
  inv = 1.0 / jnp.maximum(cnt_ref[:, 0:1], 1.0)
  t = t * inv + bl_ref[...] + r_ref[...]
  nrm = jnp.sqrt(jnp.sum(t * t, axis=1, keepdims=True))
  t = t / jnp.maximum(nrm, 1e-12)
  return jnp.where(t > 0.0, t, jnp.exp(jnp.minimum(t, 0.0)) - 1.0)


def _ep_dense_body(alo_ref, ahi_ref, cnt_ref, r_ref, bl_ref,
                   pw_ref, pb_ref, wl_ref, wr_ref,
                   ylo_ref, yhi_ref, rn_ref):
  h = _epilogue_h(alo_ref, ahi_ref, cnt_ref, r_ref, bl_ref)
  xp = jnp.maximum(
      jnp.dot(h, pw_ref[...], preferred_element_type=jnp.float32)
      + pb_ref[...], 0.0)
  y = jnp.dot(xp, wl_ref[...], preferred_element_type=jnp.float32)
  ylo_ref[...] = y[:, :_HALF]
  yhi_ref[...] = y[:, _HALF:]
  rn_ref[...] = jnp.dot(h, wr_ref[...], preferred_element_type=jnp.float32)


def _ep_dense(agg_lo, agg_hi, cnt, r, bl, pw, pb, wl, wr, bm):
  m = r.shape[0]
  row = lambda i: (i, 0)
  const = lambda i: (0, 0)
  return pl.pallas_call(
      _ep_dense_body,
      grid=(m // bm,),
      in_specs=[
          pl.BlockSpec((bm, _HALF), row),
          pl.BlockSpec((bm, _HALF), row),
          pl.BlockSpec((bm, 16), row),
          pl.BlockSpec((bm, _DH), row),
          pl.BlockSpec((1, _DH), const),
          pl.BlockSpec((_DH, _DH), const),
          pl.BlockSpec((1, _DH), const),
          pl.BlockSpec((_DH, _DH), const),
          pl.BlockSpec((_DH, _DH), const),
      ],
      out_specs=[
          pl.BlockSpec((bm, _HALF), row),
          pl.BlockSpec((bm, _HALF), row),
          pl.BlockSpec((bm, _DH), row),
      ],
      out_shape=[
          jax.ShapeDtypeStruct((m, _HALF), jnp.float32),
          jax.ShapeDtypeStruct((m, _HALF), jnp.float32),
          jax.ShapeDtypeStruct((m, _DH), jnp.float32),
      ],
      compiler_params=pltpu.CompilerParams(
          dimension_semantics=("parallel",)
      ),
  )(agg_lo, agg_hi, cnt, r, bl.reshape(1, _DH),
    pw, pb.reshape(1, _DH), wl, wr)


def _ep_head_body(alo_ref, ahi_ref, cnt_ref, r_ref, bl_ref,
                  w1_ref, b1_ref, w2_ref, b2_ref, w3_ref, b3_ref, o_ref):
  h = _epilogue_h(alo_ref, ahi_ref, cnt_ref, r_ref, bl_ref)
  for w_ref, b_ref in ((w1_ref, b1_ref), (w2_ref, b2_ref)):
    h = jnp.dot(h, w_ref[...], preferred_element_type=jnp.float32) + b_ref[...]
    h = jnp.where(h > 0.0, h, jnp.exp(jnp.minimum(h, 0.0)) - 1.0)
  o_ref[...] = (
      jnp.dot(h, w3_ref[...], preferred_element_type=jnp.float32) + b3_ref[...])


def _ep_head(agg_lo, agg_hi, cnt, r, bl, w1, b1, w2, b2, w3, b3, bm):
  m = r.shape[0]
  do_p = w3.shape[1]
  row = lambda i: (i, 0)
  const = lambda i: (0, 0)
  return pl.pallas_call(
      _ep_head_body,
      grid=(m // bm,),
      in_specs=[
          pl.BlockSpec((bm, _HALF), row),
          pl.BlockSpec((bm, _HALF), row),
          pl.BlockSpec((bm, 16), row),
          pl.BlockSpec((bm, _DH), row),
          pl.BlockSpec((1, _DH), const),
          pl.BlockSpec((_DH, _DH), const),
          pl.BlockSpec((1, _DH), const),
          pl.BlockSpec((_DH, _DH), const),
          pl.BlockSpec((1, _DH), const),
          pl.BlockSpec((_DH, do_p), const),
          pl.BlockSpec((1, do_p), const),
      ],
      out_specs=pl.BlockSpec((bm, do_p), row),
      out_shape=jax.ShapeDtypeStruct((m, do_p), jnp.float32),
      compiler_params=pltpu.CompilerParams(
          dimension_semantics=("parallel",)
      ),
  )(agg_lo, agg_hi, cnt, r, bl.reshape(1, _DH),
    w1, b1.reshape(1, _DH), w2, b2.reshape(1, _DH), w3, b3.reshape(1, do_p))


def _sc_mesh():
  return plsc.VectorSubcoreMesh(
      core_axis_name="c",
      subcore_axis_name="s",
      num_cores=_NUM_CORES,
      num_subcores=_NUM_SUBCORES,
  )


def _make_sc_agg(np_, nb):
  rows_per = np_ // _NUM_SUBCORES

  def body(y_lo, y_hi, src_t, dst_t, zrows, agg_lo, agg_hi,
           src_v, dst_v, rows_v, acc_sh, sem):
    c = lax.axis_index("c")
    s = lax.axis_index("s")
    sl = pl.ds(s * rows_per, rows_per)

    pltpu.sync_copy(zrows, acc_sh.at[sl])
    pltpu.sync_copy(src_t.at[s], src_v)
    pltpu.sync_copy(dst_t.at[s], dst_v)
    plsc.subcore_barrier()

    def run(y_ref):
      def step(j, carry):
        pltpu.async_copy(y_ref.at[src_v.at[j]], rows_v, sem).wait()
        pltpu.sync_copy(rows_v, acc_sh.at[dst_v.at[j]], add=True)
        return carry
      lax.fori_loop(0, nb, step, 0)

    @pl.when(c == 0)
    def _():
      run(y_lo)

    @pl.when(c == 1)
    def _():
      run(y_hi)

    plsc.subcore_barrier()

    @pl.when(c == 0)
    def _():
      pltpu.sync_copy(acc_sh.at[sl], agg_lo.at[sl])

    @pl.when(c == 1)
    def _():
      pltpu.sync_copy(acc_sh.at[sl], agg_hi.at[sl])

  return pl.kernel(
      body,
      out_type=[jax.ShapeDtypeStruct((np_, _HALF), jnp.float32)] * 2,
      mesh=_sc_mesh(),
      scratch_types=[
          pltpu.VMEM((nb, _EDGE_BATCH), jnp.int32),
          pltpu.VMEM((nb, _EDGE_BATCH), jnp.int32),
          pltpu.VMEM((_EDGE_BATCH, _HALF), jnp.float32),
          pltpu.VMEM_SHARED((np_, _HALF), jnp.float32),
          pltpu.SemaphoreType.DMA,
      ],
  )


def _make_sc_counts(np_, nb):
  rows_per = np_ // _NUM_SUBCORES

  def body(dst_t, zrows, ones_r, cnt, dst_v, ones_v, cnt_sh):
    c = lax.axis_index("c")
    s = lax.axis_index("s")
    sl = pl.ds(s * rows_per, rows_per)

    @pl.when(c == 0)
    def _():
      pltpu.sync_copy(zrows, cnt_sh.at[sl])
      pltpu.sync_copy(dst_t.at[s], dst_v)
      pltpu.sync_copy(ones_r, ones_v)
    plsc.subcore_barrier()

    @pl.when(c == 0)
    def _():
      def cstep(j, carry):
        pltpu.sync_copy(ones_v, cnt_sh.at[dst_v.at[j]], add=True)
        return carry
      lax.fori_loop(0, nb, cstep, 0)
    plsc.subcore_barrier()

    @pl.when(c == 0)
    def _():
      pltpu.sync_copy(cnt_sh.at[sl], cnt.at[sl])

  return pl.kernel(
      body,
      out_type=[jax.ShapeDtypeStruct((np_, _HALF), jnp.float32)],
      mesh=_sc_mesh(),
      scratch_types=[
          pltpu.VMEM((nb, _EDGE_BATCH), jnp.int32),
          pltpu.VMEM((_EDGE_BATCH, _HALF), jnp.float32),
          pltpu.VMEM_SHARED((np_, _HALF), jnp.float32),
      ],
  )


def kernel(x, edges, p1_W, p1_b, l1_Wl, l1_bl, l1_Wr,
           p2_W, p2_b, l2_Wl, l2_bl, l2_Wr,
           p3_W, p3_b, l3_Wl, l3_bl, l3_Wr,
           fc1_W, fc1_b, fc2_W, fc2_b, fc3_W, fc3_b):
  f32 = jnp.float32
  n, d_in = x.shape
  e = edges.shape[1]
  np_ = _rup(n + 1, 2048)
  dp = _rup(d_in, 128)
  d_out = fc3_W.shape[1]

  xp_in = jnp.pad(x, ((0, np_ - n), (0, dp - d_in)))
  p1_Wp = jnp.pad(p1_W, ((0, dp - d_in), (0, dp - d_in)))
  p1_bp = jnp.pad(p1_b, (0, dp - d_in))
  l1_Wlp = jnp.pad(l1_Wl, ((0, dp - d_in), (0, 0)))
  l1_Wrp = jnp.pad(l1_Wr, ((0, dp - d_in), (0, 0)))

  chunk = _NUM_SUBCORES * _EDGE_BATCH
  nb = _rup(e, chunk) // chunk
  ep = nb * chunk
  src = jnp.concatenate([edges[0], jnp.zeros((ep - e,), jnp.int32)])
  dst = jnp.concatenate([edges[1], jnp.full((ep - e,), n, jnp.int32)])
  src_t = src.reshape(_NUM_SUBCORES, nb, _EDGE_BATCH)
  dst_t = dst.reshape(_NUM_SUBCORES, nb, _EDGE_BATCH)

  rows_per = np_ // _NUM_SUBCORES
  zrows = jnp.zeros((rows_per, _HALF), f32)
  ones_r = jnp.ones((_EDGE_BATCH, _HALF), f32)

  sc_agg = _make_sc_agg(np_, nb)
  sc_counts = _make_sc_counts(np_, nb)

  cnt = sc_counts(dst_t, zrows, ones_r)
  if isinstance(cnt, (list, tuple)):
    cnt = cnt[0]
  cnt = cnt[:, :16]
  y_lo, y_hi, r = _l1_dense(xp_in, p1_Wp, p1_bp, l1_Wlp, l1_Wrp, bm=512)
  agg_lo, agg_hi = sc_agg(y_lo, y_hi, src_t, dst_t, zrows)

  for pw, pb, wl, wr, bl in (
      (p2_W, p2_b, l2_Wl, l2_Wr, l1_bl),
      (p3_W, p3_b, l3_Wl, l3_Wr, l2_bl),
  ):
    y_lo, y_hi, r = _ep_dense(
        agg_lo, agg_hi, cnt, r, bl, pw, pb, wl, wr, bm=1024)
    agg_lo, agg_hi = sc_agg(y_lo, y_hi, src_t, dst_t, zrows)

  do_p = _rup(d_out, 128)
  fc3_Wp = jnp.pad(fc3_W, ((0, 0), (0, do_p - d_out)))
  fc3_bp = jnp.pad(fc3_b, (0, do_p - d_out))
  out = _ep_head(agg_lo, agg_hi, cnt, r, l3_bl,
                 fc1_W, fc1_b, fc2_W, fc2_b, fc3_Wp, fc3_bp, bm=1024)
  return out[:n, :d_out]


# speedup vs baseline: 6.7946x; 1.2610x over previous
import functools

import jax
import jax.numpy as jnp
from jax import lax
from jax.experimental import pallas as pl
from jax.experimental.pallas import tpu as pltpu
from jax.experimental.pallas import tpu_sc as plsc

_NUM_CORES = 2
_NUM_SUBCORES = 16
_EDGE_BATCH = 128
_DH = 256
_HALF = 128


def _rup(v, m):
  return (v + m - 1) // m * m


def _l1_body(x_ref, w_ref, b_ref, wl_ref, wr_ref,
             ylo_ref, yhi_ref, r_ref, xp_ref):
  dp = w_ref.shape[1]
  kt = 896 if dp % 896 == 0 else dp
  for t in range(dp // kt):
    sl = slice(t * kt, (t + 1) * kt)
    xp_ref[:, sl] = jnp.maximum(
        jnp.dot(x_ref[...], w_ref[:, sl],
                preferred_element_type=jnp.float32) + b_ref[:, sl], 0.0)
  y = jnp.dot(xp_ref[...], wl_ref[...], preferred_element_type=jnp.float32)
  ylo_ref[...] = y[:, :_HALF]
  yhi_ref[...] = y[:, _HALF:]
  r_ref[...] = jnp.dot(x_ref[...], wr_ref[...],
                       preferred_element_type=jnp.float32)


def _l1_dense(x, w, b, wl, wr, bm):
  m, d_in = x.shape
  dp = w.shape[1]
  bm = min(bm, m)
  assert m % bm == 0, (m, bm)
  return pl.pallas_call(
      _l1_body,
      grid=(m // bm,),
      in_specs=[
          pl.BlockSpec((bm, d_in), lambda i: (i, 0)),
          pl.BlockSpec((d_in, dp), lambda i: (0, 0)),
          pl.BlockSpec((1, dp), lambda i: (0, 0)),
          pl.BlockSpec((dp, _DH), lambda i: (0, 0)),
          pl.BlockSpec((d_in, _DH), lambda i: (0, 0)),
      ],
      out_specs=[
          pl.BlockSpec((bm, _HALF), lambda i: (i, 0)),
          pl.BlockSpec((bm, _HALF), lambda i: (i, 0)),
          pl.BlockSpec((bm, _DH), lambda i: (i, 0)),
      ],
      out_shape=[
          jax.ShapeDtypeStruct((m, _HALF), jnp.float32),
          jax.ShapeDtypeStruct((m, _HALF), jnp.float32),
          jax.ShapeDtypeStruct((m, _DH), jnp.float32),
      ],
      scratch_shapes=[pltpu.VMEM((bm, dp), jnp.float32)],
      compiler_params=pltpu.CompilerParams(
          dimension_semantics=("parallel",)
      ),
  )(x, w, b.reshape(1, dp), wl, wr)


def _epilogue_h(alo_ref, ahi_ref, cnt_ref, r_ref, bl_ref):
  t = jnp.concatenate([alo_ref[...], ahi_ref[...]], axis=1)
  inv = 1.0 / jnp.maximum(cnt_ref[:, 0:1], 1.0)
  t = t * inv + bl_ref[...] + r_ref[...]
  nrm = jnp.sqrt(jnp.sum(t * t, axis=1, keepdims=True))
  t = t / jnp.maximum(nrm, 1e-12)
  return jnp.where(t > 0.0, t, jnp.exp(jnp.minimum(t, 0.0)) - 1.0)


def _ep_dense_body(alo_ref, ahi_ref, cnt_ref, r_ref, bl_ref,
                   pw_ref, pb_ref, wl_ref, wr_ref,
                   ylo_ref, yhi_ref, rn_ref):
  h = _epilogue_h(alo_ref, ahi_ref, cnt_ref, r_ref, bl_ref)
  xp = jnp.maximum(
      jnp.dot(h, pw_ref[...], preferred_element_type=jnp.float32)
      + pb_ref[...], 0.0)
  y = jnp.dot(xp, wl_ref[...], preferred_element_type=jnp.float32)
  ylo_ref[...] = y[:, :_HALF]
  yhi_ref[...] = y[:, _HALF:]
  rn_ref[...] = jnp.dot(h, wr_ref[...], preferred_element_type=jnp.float32)


def _ep_dense(agg_lo, agg_hi, cnt, r, bl, pw, pb, wl, wr, bm):
  m = r.shape[0]
  bm = min(bm, m)
  assert m % bm == 0, (m, bm)
  row = lambda i: (i, 0)
  const = lambda i: (0, 0)
  return pl.pallas_call(
      _ep_dense_body,
      grid=(m // bm,),
      in_specs=[
          pl.BlockSpec((bm, _HALF), row),
          pl.BlockSpec((bm, _HALF), row),
          pl.BlockSpec((bm, 16), row),
          pl.BlockSpec((bm, _DH), row),
          pl.BlockSpec((1, _DH), const),
          pl.BlockSpec((_DH, _DH), const),
          pl.BlockSpec((1, _DH), const),
          pl.BlockSpec((_DH, _DH), const),
          pl.BlockSpec((_DH, _DH), const),
      ],
      out_specs=[
          pl.BlockSpec((bm, _HALF), row),
          pl.BlockSpec((bm, _HALF), row),
          pl.BlockSpec((bm, _DH), row),
      ],
      out_shape=[
          jax.ShapeDtypeStruct((m, _HALF), jnp.float32),
          jax.ShapeDtypeStruct((m, _HALF), jnp.float32),
          jax.ShapeDtypeStruct((m, _DH), jnp.float32),
      ],
      compiler_params=pltpu.CompilerParams(
          dimension_semantics=("parallel",)
      ),
  )(agg_lo, agg_hi, cnt, r, bl.reshape(1, _DH),
    pw, pb.reshape(1, _DH), wl, wr)


def _ep_head_body(alo_ref, ahi_ref, cnt_ref, r_ref, bl_ref,
                  w1_ref, b1_ref, w2_ref, b2_ref, w3_ref, b3_ref, o_ref):
  h = _epilogue_h(alo_ref, ahi_ref, cnt_ref, r_ref, bl_ref)
  for w_ref, b_ref in ((w1_ref, b1_ref), (w2_ref, b2_ref)):
    h = jnp.dot(h, w_ref[...], preferred_element_type=jnp.float32) + b_ref[...]
    h = jnp.where(h > 0.0, h, jnp.exp(jnp.minimum(h, 0.0)) - 1.0)
  o_ref[...] = (
      jnp.dot(h, w3_ref[...], preferred_element_type=jnp.float32) + b3_ref[...])


def _ep_head(agg_lo, agg_hi, cnt, r, bl, w1, b1, w2, b2, w3, b3, bm):
  m = r.shape[0]
  bm = min(bm, m)
  assert m % bm == 0, (m, bm)
  do_p = w3.shape[1]
  row = lambda i: (i, 0)
  const = lambda i: (0, 0)
  return pl.pallas_call(
      _ep_head_body,
      grid=(m // bm,),
      in_specs=[
          pl.BlockSpec((bm, _HALF), row),
          pl.BlockSpec((bm, _HALF), row),
          pl.BlockSpec((bm, 16), row),
          pl.BlockSpec((bm, _DH), row),
          pl.BlockSpec((1, _DH), const),
          pl.BlockSpec((_DH, _DH), const),
          pl.BlockSpec((1, _DH), const),
          pl.BlockSpec((_DH, _DH), const),
          pl.BlockSpec((1, _DH), const),
          pl.BlockSpec((_DH, do_p), const),
          pl.BlockSpec((1, do_p), const),
      ],
      out_specs=pl.BlockSpec((bm, do_p), row),
      out_shape=jax.ShapeDtypeStruct((m, do_p), jnp.float32),
      compiler_params=pltpu.CompilerParams(
          dimension_semantics=("parallel",)
      ),
  )(agg_lo, agg_hi, cnt, r, bl.reshape(1, _DH),
    w1, b1.reshape(1, _DH), w2, b2.reshape(1, _DH), w3, b3.reshape(1, do_p))


def _sc_mesh():
  return plsc.VectorSubcoreMesh(
      core_axis_name="c",
      subcore_axis_name="s",
      num_cores=_NUM_CORES,
      num_subcores=_NUM_SUBCORES,
  )


_GB = 8


def _make_sc_agg(np_, nb):
  rows_per = np_ // _NUM_SUBCORES
  ng = nb // _GB

  def body(y_lo, y_hi, src_t, dst_t, zrows, agg_lo, agg_hi,
           src_c, dst_c, rows_v, acc_sh, sem_i, sem_a, sem_b):
    c = lax.axis_index("c")
    s = lax.axis_index("s")
    sl = pl.ds(s * rows_per, rows_per)

    pltpu.sync_copy(zrows, acc_sh.at[sl])
    plsc.subcore_barrier()

    def run(y_ref):
      sems = (sem_a, sem_b)

      def idx_start(g, slot):
        off = g * _GB
        pltpu.async_copy(src_t.at[s, pl.ds(off, _GB)], src_c.at[slot], sem_i)
        pltpu.async_copy(dst_t.at[s, pl.ds(off, _GB)], dst_c.at[slot], sem_i)

      def idx_wait(slot):
        pltpu.make_async_copy(
            src_t.at[s, pl.ds(0, _GB)], src_c.at[slot], sem_i).wait()
        pltpu.make_async_copy(
            dst_t.at[s, pl.ds(0, _GB)], dst_c.at[slot], sem_i).wait()

      idx_start(0, 0)
      if ng > 1:
        idx_start(1, 1)
      idx_wait(0)
      pltpu.async_copy(y_ref.at[src_c.at[0, 0]], rows_v.at[0], sem_a)

      def group(g, carry):
        p = g % 2
        for b in range(_GB):
          buf = b % 2
          if b + 1 < _GB:
            pltpu.async_copy(
                y_ref.at[src_c.at[p, b + 1]], rows_v.at[1 - buf],
                sems[1 - buf])
          pltpu.make_async_copy(
              y_ref.at[src_c.at[p, b]], rows_v.at[buf], sems[buf]).wait()
          pltpu.sync_copy(rows_v.at[buf], acc_sh.at[dst_c.at[p, b]], add=True)

        @pl.when(g + 2 < ng)
        def _():
          idx_start(g + 2, p)

        @pl.when(g + 1 < ng)
        def _():
          idx_wait(1 - p)
          pltpu.async_copy(
              y_ref.at[src_c.at[1 - p, 0]], rows_v.at[0], sem_a)

        return carry
      lax.fori_loop(0, ng, group, 0)

    @pl.when(c == 0)
    def _():
      run(y_lo)

    @pl.when(c == 1)
    def _():
      run(y_hi)

    plsc.subcore_barrier()

    @pl.when(c == 0)
    def _():
      pltpu.sync_copy(acc_sh.at[sl], agg_lo.at[sl])

    @pl.when(c == 1)
    def _():
      pltpu.sync_copy(acc_sh.at[sl], agg_hi.at[sl])

  return pl.kernel(
      body,
      out_type=[jax.ShapeDtypeStruct((np_, _HALF), jnp.float32)] * 2,
      mesh=_sc_mesh(),
      scratch_types=[
          pltpu.VMEM((2, _GB, _EDGE_BATCH), jnp.int32),
          pltpu.VMEM((2, _GB, _EDGE_BATCH), jnp.int32),
          pltpu.VMEM((2, _EDGE_BATCH, _HALF), jnp.float32),
          pltpu.VMEM_SHARED((np_, _HALF), jnp.float32),
          pltpu.SemaphoreType.DMA,
          pltpu.SemaphoreType.DMA,
          pltpu.SemaphoreType.DMA,
      ],
  )


def _make_sc_counts(np_, nb):
  rows_per = np_ // _NUM_SUBCORES

  def body(dst_t, zrows, ones_r, cnt, dst_v, ones_v, cnt_sh):
    c = lax.axis_index("c")
    s = lax.axis_index("s")
    sl = pl.ds(s * rows_per, rows_per)

    @pl.when(c == 0)
    def _():
      pltpu.sync_copy(zrows, cnt_sh.at[sl])
      pltpu.sync_copy(dst_t.at[s], dst_v)
      pltpu.sync_copy(ones_r, ones_v)
    plsc.subcore_barrier()

    @pl.when(c == 0)
    def _():
      def cstep(j, carry):
        pltpu.sync_copy(ones_v, cnt_sh.at[dst_v.at[j]], add=True)
        return carry
      lax.fori_loop(0, nb, cstep, 0)
    plsc.subcore_barrier()

    @pl.when(c == 0)
    def _():
      pltpu.sync_copy(cnt_sh.at[sl], cnt.at[sl])

  return pl.kernel(
      body,
      out_type=[jax.ShapeDtypeStruct((np_, _HALF), jnp.float32)],
      mesh=_sc_mesh(),
      scratch_types=[
          pltpu.VMEM((nb, _EDGE_BATCH), jnp.int32),
          pltpu.VMEM((_EDGE_BATCH, _HALF), jnp.float32),
          pltpu.VMEM_SHARED((np_, _HALF), jnp.float32),
      ],
  )


def kernel(x, edges, p1_W, p1_b, l1_Wl, l1_bl, l1_Wr,
           p2_W, p2_b, l2_Wl, l2_bl, l2_Wr,
           p3_W, p3_b, l3_Wl, l3_bl, l3_Wr,
           fc1_W, fc1_b, fc2_W, fc2_b, fc3_W, fc3_b):
  f32 = jnp.float32
  n, d_in = x.shape
  e = edges.shape[1]
  np_ = _rup(n + 1, 2048)
  dp = _rup(d_in, 128)
  d_out = fc3_W.shape[1]

  p1_Wp = jnp.pad(p1_W, ((0, 0), (0, dp - d_in)))
  p1_bp = jnp.pad(p1_b, (0, dp - d_in))
  l1_Wlp = jnp.pad(l1_Wl, ((0, dp - d_in), (0, 0)))

  chunk = _NUM_SUBCORES * _EDGE_BATCH
  nb = _rup(_rup(e, chunk) // chunk, _GB)
  ep = nb * chunk
  src = jnp.concatenate([edges[0], jnp.zeros((ep - e,), jnp.int32)])
  dst = jnp.concatenate([edges[1], jnp.full((ep - e,), n, jnp.int32)])
  src_t = src.reshape(_NUM_SUBCORES, nb, _EDGE_BATCH)
  dst_t = dst.reshape(_NUM_SUBCORES, nb, _EDGE_BATCH)

  rows_per = np_ // _NUM_SUBCORES
  zrows = jnp.zeros((rows_per, _HALF), f32)
  ones_r = jnp.ones((_EDGE_BATCH, _HALF), f32)

  sc_agg = _make_sc_agg(np_, nb)
  sc_counts = _make_sc_counts(np_, nb)

  cnt = sc_counts(dst_t, zrows, ones_r)
  if isinstance(cnt, (list, tuple)):
    cnt = cnt[0]
  cnt = cnt[:, :16]
  y_lo, y_hi, r = _l1_dense(x, p1_Wp, p1_bp, l1_Wlp, l1_Wr, bm=400)
  agg_lo, agg_hi = sc_agg(y_lo, y_hi, src_t, dst_t, zrows)

  for pw, pb, wl, wr, bl in (
      (p2_W, p2_b, l2_Wl, l2_Wr, l1_bl),
      (p3_W, p3_b, l3_Wl, l3_Wr, l2_bl),
  ):
    y_lo, y_hi, r = _ep_dense(
        agg_lo, agg_hi, cnt, r, bl, pw, pb, wl, wr, bm=2000)
    agg_lo, agg_hi = sc_agg(y_lo, y_hi, src_t, dst_t, zrows)

  do_p = _rup(d_out, 128)
  fc3_Wp = jnp.pad(fc3_W, ((0, 0), (0, do_p - d_out)))
  fc3_bp = jnp.pad(fc3_b, (0, do_p - d_out))
  out = _ep_head(agg_lo, agg_hi, cnt, r, l3_bl,
                 fc1_W, fc1_b, fc2_W, fc2_b, fc3_Wp, fc3_bp, bm=2000)
  return out[:n, :d_out]
